# Initial kernel scaffold; baseline (speedup 1.0000x reference)
import jax
import jax.numpy as jnp
from jax.experimental import pallas as pl


def kernel(hidden_states, gate_weight, expert_gate_w, expert_up_w, expert_down_w, shared_gate_w, shared_up_w, shared_down_w):
    raise NotImplementedError("write your pallas kernel here")


# speedup vs baseline: 4.4209x; 4.4209x over previous
import functools

import jax
import jax.numpy as jnp
from jax import lax
from jax.experimental import pallas as pl
from jax.experimental.pallas import tpu as pltpu

N, H, E, K, I = 2048, 1024, 16, 2, 512
NEG = -1e30
INTMAX = 2147483647


def _mlp_bf16(xb, gw, uw, dw):
    g = lax.dot_general(xb, gw, (((1,), (1,)), ((), ())),
                        preferred_element_type=jnp.float32)
    u = lax.dot_general(xb, uw, (((1,), (1,)), ((), ())),
                        preferred_element_type=jnp.float32)
    act = 0.5 * g * (1.0 + lax.erf(g * 0.7071067811865476))
    h = (act * u).astype(jnp.bfloat16)
    return lax.dot_general(h, dw, (((1,), (1,)), ((), ())),
                           preferred_element_type=jnp.float32)


def _body(x_ref, xb_ref, gw_ref, eg_ref, eu_ref, ed_ref, sg_ref, su_ref,
          sd_ref, out_ref, acc, w1r, w2r, a1r, a2r):
    s = pl.program_id(0)

    @pl.when(s == 0)
    def _gate():
        logits = lax.dot_general(x_ref[...], gw_ref[...], (((1,), (1,)), ((), ())),
                                 preferred_element_type=jnp.float32)
        iota = lax.broadcasted_iota(jnp.int32, (N, 128), 1)
        l = jnp.where(iota < E, logits, NEG)
        m = jnp.max(l, axis=1, keepdims=True)
        z = jnp.sum(jnp.exp(l - m), axis=1, keepdims=True)
        m1 = jnp.max(l, axis=1, keepdims=True)
        a1 = jnp.min(jnp.where(l == m1, iota, INTMAX), axis=1, keepdims=True)
        l2 = jnp.where(iota == a1, NEG, l)
        m2 = jnp.max(l2, axis=1, keepdims=True)
        a2 = jnp.min(jnp.where(l2 == m2, iota, INTMAX), axis=1, keepdims=True)
        s1 = jnp.exp(m1 - m) / z
        s2 = jnp.exp(m2 - m) / z
        denom = s1 + s2 + 1e-20
        w1r[...] = s1 / denom
        w2r[...] = s2 / denom
        a1r[...] = a1
        a2r[...] = a2
        acc[...] = jnp.zeros((N, H), jnp.float32)

    @pl.when((s >= 1) & (s <= E))
    def _expert():
        e = s - 1
        coef = (w1r[...] * (a1r[...] == e).astype(jnp.float32)
                + w2r[...] * (a2r[...] == e).astype(jnp.float32))
        y = _mlp_bf16(xb_ref[...], eg_ref[0], eu_ref[0], ed_ref[0])
        acc[...] += coef * y

    @pl.when(s == E + 1)
    def _shared():
        y = _mlp_bf16(xb_ref[...], sg_ref[...], su_ref[...], sd_ref[...])
        out_ref[...] = acc[...] + y


@jax.jit
def _moe(x, gwp, eg, eu, ed, sg, su, sd):
    xb = x.astype(jnp.bfloat16)
    return pl.pallas_call(
        _body,
        grid=(E + 2,),
        in_specs=[
            pl.BlockSpec((N, H), lambda s: (0, 0)),
            pl.BlockSpec((N, H), lambda s: (0, 0)),
            pl.BlockSpec((128, H), lambda s: (0, 0)),
            pl.BlockSpec((1, I, H), lambda s: (jnp.clip(s - 1, 0, E - 1), 0, 0)),
            pl.BlockSpec((1, I, H), lambda s: (jnp.clip(s - 1, 0, E - 1), 0, 0)),
            pl.BlockSpec((1, H, I), lambda s: (jnp.clip(s - 1, 0, E - 1), 0, 0)),
            pl.BlockSpec((I, H), lambda s: (0, 0)),
            pl.BlockSpec((I, H), lambda s: (0, 0)),
            pl.BlockSpec((H, I), lambda s: (0, 0)),
        ],
        out_specs=pl.BlockSpec((N, H), lambda s: (0, 0)),
        out_shape=jax.ShapeDtypeStruct((N, H), jnp.float32),
        scratch_shapes=[
            pltpu.VMEM((N, H), jnp.float32),
            pltpu.VMEM((N, 1), jnp.float32),
            pltpu.VMEM((N, 1), jnp.float32),
            pltpu.VMEM((N, 1), jnp.int32),
            pltpu.VMEM((N, 1), jnp.int32),
        ],
        compiler_params=pltpu.CompilerParams(
            dimension_semantics=("arbitrary",),
        ),
    )(x, xb, gwp, eg, eu, ed, sg, su, sd)


def kernel(hidden_states, gate_weight, expert_gate_w, expert_up_w,
           expert_down_w, shared_gate_w, shared_up_w, shared_down_w):
    b, s, h = hidden_states.shape
    x = hidden_states.reshape(-1, h).astype(jnp.float32)
    gwp = jnp.zeros((128, h), jnp.float32).at[:E].set(gate_weight)
    out = _moe(x, gwp,
               expert_gate_w.astype(jnp.bfloat16),
               expert_up_w.astype(jnp.bfloat16),
               expert_down_w.astype(jnp.bfloat16),
               shared_gate_w.astype(jnp.bfloat16),
               shared_up_w.astype(jnp.bfloat16),
               shared_down_w.astype(jnp.bfloat16))
    return out.reshape(b, s, h)
